# Initial kernel scaffold; baseline (speedup 1.0000x reference)
import jax
import jax.numpy as jnp
from jax.experimental import pallas as pl


def kernel(feature, proposal, W1, b1, Wcls, bcls, Wreg, breg, image_shape):
    raise NotImplementedError("write your pallas kernel here")


# speedup vs baseline: 5.4527x; 5.4527x over previous
import jax
import jax.numpy as jnp
from jax.experimental import pallas as pl
from jax.experimental.pallas import tpu as pltpu

_NUM_CLASSES = 21
_OUT = 7
_SCALE = 1.0 / 16.0
_SCORE_TH = 0.05
_NMS_TH = 0.5
_NUM_DET = 50
_MIN_SIZE = 1.0
_BBOX_CLIP = 4.135166556742356
_H = 50
_W = 50
_C = 256
_NB = 128
_NPAD = 1024
_NHEAD = _NUM_CLASSES * 5
_P = 2560
_LPAD = 32
_K = _C * _OUT * _OUT


def _pool_kernel(prop_ref, f_ref, yk_ref, xk_ref, out_ref):
    ij = pl.program_id(0)
    i = ij // _OUT
    j = ij - i * _OUT
    gi = (i.astype(jnp.float32) + 0.5) / _OUT
    gj = (j.astype(jnp.float32) + 0.5) / _OUT

    b = prop_ref[...] * _SCALE
    x1 = b[:, 0:1]
    y1 = b[:, 1:2]
    x2 = b[:, 2:3]
    y2 = b[:, 3:4]
    x = jnp.clip(x1 + gj * (x2 - x1), 0.0, _W - 1.0)
    y = jnp.clip(y1 + gi * (y2 - y1), 0.0, _H - 1.0)
    x0 = jnp.clip(jnp.floor(x), 0.0, _W - 2.0)
    y0 = jnp.clip(jnp.floor(y), 0.0, _H - 2.0)
    fx = x - x0
    fy = y - y0
    x0i = x0.astype(jnp.int32)
    y0i = y0.astype(jnp.int32)

    yk = yk_ref[...]
    xk = xk_ref[...]
    one = jnp.float32(1.0)
    zero = jnp.float32(0.0)

    def sel(dy, dx):
        s = jnp.where((yk == y0i + dy) & (xk == x0i + dx), one, zero)
        return jax.lax.dot_general(s, f_ref[...], (((1,), (0,)), ((), ())),
                                   preferred_element_type=jnp.float32,
                                   precision=jax.lax.Precision.HIGHEST)

    g00 = sel(0, 0)
    g01 = sel(0, 1)
    g10 = sel(1, 0)
    g11 = sel(1, 1)
    out_ref[0] = (g00 * (1.0 - fy) * (1.0 - fx)
                  + g01 * (1.0 - fy) * fx
                  + g10 * fy * (1.0 - fx)
                  + g11 * fy * fx)


def _fc_kernel(x_ref, w1_ref, b1_ref, wc_ref, bc_ref, out_ref):
    h = jax.lax.dot_general(x_ref[...], w1_ref[...], (((1,), (0,)), ((), ())),
                            preferred_element_type=jnp.float32,
                            precision=jax.lax.Precision.DEFAULT)
    h = jnp.maximum(h + b1_ref[...], 0.0)
    out_ref[...] = jax.lax.dot_general(
        h, wc_ref[...], (((1,), (0,)), ((), ())),
        preferred_element_type=jnp.float32,
        precision=jax.lax.Precision.DEFAULT) + bc_ref[...]


def _heads(feature, proposal, W1, b1, Wcls, bcls, Wreg, breg):
    f2 = jnp.zeros((_P, _C), jnp.float32).at[:_H * _W].set(
        feature.transpose(1, 2, 0).reshape(_H * _W, _C))
    prop_pad = jnp.zeros((_NPAD, 4), jnp.float32).at[:1000].set(proposal)
    pcol = jnp.arange(_P, dtype=jnp.int32)
    sentinel = jnp.int32(-1000)
    yk = jnp.where(pcol < _H * _W, pcol // _W, sentinel).reshape(1, _P)
    xk = jnp.where(pcol < _H * _W, pcol % _W, sentinel).reshape(1, _P)

    pooled = pl.pallas_call(
        _pool_kernel,
        grid=(_OUT * _OUT, _NPAD // _NB),
        in_specs=[
            pl.BlockSpec((_NB, 4), lambda ij, k: (k, 0)),
            pl.BlockSpec((_P, _C), lambda ij, k: (0, 0)),
            pl.BlockSpec((1, _P), lambda ij, k: (0, 0)),
            pl.BlockSpec((1, _P), lambda ij, k: (0, 0)),
        ],
        out_specs=pl.BlockSpec((1, _NB, _C), lambda ij, k: (ij, k, 0)),
        out_shape=jax.ShapeDtypeStruct((_OUT * _OUT, _NPAD, _C), jnp.float32),
    )(prop_pad, f2, yk, xk)

    xmat = pooled.transpose(1, 2, 0).reshape(_NPAD, _K)
    wcat = jnp.concatenate([Wcls, Wreg], axis=1)
    bcat = jnp.concatenate([bcls, breg]).reshape(1, _NHEAD)

    out = pl.pallas_call(
        _fc_kernel,
        grid=(_NPAD // _NB,),
        in_specs=[
            pl.BlockSpec((_NB, _K), lambda k: (k, 0)),
            pl.BlockSpec((_K, 1024), lambda k: (0, 0)),
            pl.BlockSpec((1, 1024), lambda k: (0, 0)),
            pl.BlockSpec((1024, _NHEAD), lambda k: (0, 0)),
            pl.BlockSpec((1, _NHEAD), lambda k: (0, 0)),
        ],
        out_specs=pl.BlockSpec((_NB, _NHEAD), lambda k: (k, 0)),
        out_shape=jax.ShapeDtypeStruct((_NPAD, _NHEAD), jnp.float32),
        compiler_params=pltpu.CompilerParams(
            vmem_limit_bytes=100 * 1024 * 1024),
    )(xmat, W1, b1.reshape(1, 1024), wcat, bcat)
    return out[:1000]


def _nms_kernel(lg_ref, dx_ref, dy_ref, dw_ref, dh_ref, prop_ref, img_ref,
                x1_ref, y1_ref, x2_ref, y2_ref, s_ref, l_ref):
    L, N = lg_ref.shape
    hm1 = img_ref[0:1, 0:1] - 1.0
    wm1 = img_ref[0:1, 1:2] - 1.0

    lg = lg_ref[...]
    m = jnp.max(lg, axis=0, keepdims=True)
    e = jnp.exp(lg - m)
    score = e / jnp.sum(e, axis=0, keepdims=True)

    dx = dx_ref[...] * 0.1
    dy = dy_ref[...] * 0.1
    dw = jnp.minimum(dw_ref[...] * 0.2, _BBOX_CLIP)
    dh = jnp.minimum(dh_ref[...] * 0.2, _BBOX_CLIP)

    px1 = prop_ref[0:1, :]
    py1 = prop_ref[1:2, :]
    px2 = prop_ref[2:3, :]
    py2 = prop_ref[3:4, :]
    w = px2 - px1
    h = py2 - py1
    cx = px1 + 0.5 * w
    cy = py1 + 0.5 * h
    pcx = dx * w + cx
    pcy = dy * h + cy
    pw = jnp.exp(dw) * w
    ph = jnp.exp(dh) * h
    bx1 = jnp.minimum(jnp.maximum(pcx - 0.5 * pw, 0.0), wm1)
    by1 = jnp.minimum(jnp.maximum(pcy - 0.5 * ph, 0.0), hm1)
    bx2 = jnp.minimum(jnp.maximum(pcx + 0.5 * pw, 0.0), wm1)
    by2 = jnp.minimum(jnp.maximum(pcy + 0.5 * ph, 0.0), hm1)
    ws = bx2 - bx1
    hs = by2 - by1
    areas = ws * hs
    lane = jax.lax.broadcasted_iota(jnp.int32, (L, N), 1)
    valid = ((ws >= _MIN_SIZE) & (hs >= _MIN_SIZE) & (score >= _SCORE_TH)
             & (lane < 1000))
    neg = jnp.float32(-1e9)
    s0 = jnp.where(valid, score, neg)

    col = jax.lax.broadcasted_iota(jnp.int32, (L, _NUM_DET), 1)
    zer = jnp.zeros((L, _NUM_DET), jnp.float32)

    def body(t, carry):
        s, os_, ox1, oy1, ox2, oy2 = carry
        mt = jnp.max(s, axis=1, keepdims=True)
        idx = jnp.min(jnp.where(s == mt, lane, jnp.int32(2**30)),
                      axis=1, keepdims=True)
        oh = lane == idx
        sx1 = jnp.sum(jnp.where(oh, bx1, 0.0), axis=1, keepdims=True)
        sy1 = jnp.sum(jnp.where(oh, by1, 0.0), axis=1, keepdims=True)
        sx2 = jnp.sum(jnp.where(oh, bx2, 0.0), axis=1, keepdims=True)
        sy2 = jnp.sum(jnp.where(oh, by2, 0.0), axis=1, keepdims=True)
        xx1 = jnp.maximum(sx1, bx1)
        yy1 = jnp.maximum(sy1, by1)
        xx2 = jnp.minimum(sx2, bx2)
        yy2 = jnp.minimum(sy2, by2)
        inter = (jnp.maximum(xx2 - xx1, 0.0) * jnp.maximum(yy2 - yy1, 0.0))
        ai = (sx2 - sx1) * (sy2 - sy1)
        iou = inter / (ai + areas - inter + 1e-9)
        s = jnp.where(iou > _NMS_TH, neg, s)
        s = jnp.where(oh, neg, s)
        cm = col == t
        os_ = jnp.where(cm, mt, os_)
        ox1 = jnp.where(cm, sx1, ox1)
        oy1 = jnp.where(cm, sy1, oy1)
        ox2 = jnp.where(cm, sx2, ox2)
        oy2 = jnp.where(cm, sy2, oy2)
        return s, os_, ox1, oy1, ox2, oy2

    s, os_, ox1, oy1, ox2, oy2 = jax.lax.fori_loop(
        0, _NUM_DET, body, (s0, zer, zer, zer, zer, zer))

    v = os_ >= _SCORE_TH
    rowl = jax.lax.broadcasted_iota(jnp.int32, (L, _NUM_DET), 0)
    x1_ref[...] = jnp.where(v, ox1, 0.0)
    y1_ref[...] = jnp.where(v, oy1, 0.0)
    x2_ref[...] = jnp.where(v, ox2, 0.0)
    y2_ref[...] = jnp.where(v, oy2, 0.0)
    s_ref[...] = jnp.where(v, os_, 0.0)
    l_ref[...] = jnp.where(v, rowl, 0)


def kernel(feature, proposal, W1, b1, Wcls, bcls, Wreg, breg, image_shape):
    out = _heads(feature, proposal, W1, b1, Wcls, bcls, Wreg, breg)

    def padLN(a, fill=0.0):
        return jnp.full((_LPAD, _NPAD), fill, jnp.float32).at[
            :_NUM_CLASSES, :1000].set(a)

    logitT = padLN(out[:, :_NUM_CLASSES].T, -1e5)
    dxT = padLN(out[:, _NUM_CLASSES + 0::4].T)
    dyT = padLN(out[:, _NUM_CLASSES + 1::4].T)
    dwT = padLN(out[:, _NUM_CLASSES + 2::4].T)
    dhT = padLN(out[:, _NUM_CLASSES + 3::4].T)
    propT = jnp.zeros((8, _NPAD), jnp.float32).at[:4, :1000].set(proposal.T)
    imgf = jnp.asarray(image_shape, jnp.float32).reshape(1, 2)

    shp = (_LPAD, _NUM_DET)
    x1o, y1o, x2o, y2o, so, lo = pl.pallas_call(
        _nms_kernel,
        out_shape=[
            jax.ShapeDtypeStruct(shp, jnp.float32),
            jax.ShapeDtypeStruct(shp, jnp.float32),
            jax.ShapeDtypeStruct(shp, jnp.float32),
            jax.ShapeDtypeStruct(shp, jnp.float32),
            jax.ShapeDtypeStruct(shp, jnp.float32),
            jax.ShapeDtypeStruct(shp, jnp.int32),
        ],
    )(logitT, dxT, dyT, dwT, dhT, propT, imgf)

    boxes = jnp.stack([x1o, y1o, x2o, y2o], axis=-1)[1:_NUM_CLASSES]
    return boxes, so[1:_NUM_CLASSES], lo[1:_NUM_CLASSES]


# speedup vs baseline: 12.3029x; 2.2563x over previous
import functools

import jax
import jax.numpy as jnp
from jax.experimental import pallas as pl
from jax.experimental.pallas import tpu as pltpu
from jax.experimental.pallas import tpu_sc as plsc

_NUM_CLASSES = 21
_OUT = 7
_SCALE = 1.0 / 16.0
_SCORE_TH = 0.05
_NMS_TH = 0.5
_NUM_DET = 50
_MIN_SIZE = 1.0
_BBOX_CLIP = 4.135166556742356
_H = 50
_W = 50
_C = 256
_NB = 128
_NPAD = 1024
_NHEAD = _NUM_CLASSES * 5
_P = 2560
_LPAD = 32
_K = _C * _OUT * _OUT


def _idx_kernel(prop_ref, rx_ref, fx_ref, fy_ref):
    ij = pl.program_id(0)
    i = ij // _OUT
    j = ij - i * _OUT
    gi = (i.astype(jnp.float32) + 0.5) / _OUT
    gj = (j.astype(jnp.float32) + 0.5) / _OUT

    b = prop_ref[...] * _SCALE
    x1 = b[:, 0:1]
    y1 = b[:, 1:2]
    x2 = b[:, 2:3]
    y2 = b[:, 3:4]
    x = jnp.clip(x1 + gj * (x2 - x1), 0.0, _W - 1.0)
    y = jnp.clip(y1 + gi * (y2 - y1), 0.0, _H - 1.0)
    x0 = jnp.clip(jnp.floor(x), 0.0, _W - 2.0)
    y0 = jnp.clip(jnp.floor(y), 0.0, _H - 2.0)
    x0i = x0.astype(jnp.int32)
    y0i = y0.astype(jnp.int32)
    rx_ref[0] = y0i * _W + x0i
    fx_ref[0] = x - x0
    fy_ref[0] = y - y0


_NROW = _OUT * _OUT * _NPAD
_NW = 32
_RPW = _NROW // _NW
_CH = 32


def _sc_gather(ridx, fquad):
    mesh = plsc.VectorSubcoreMesh(core_axis_name="c", subcore_axis_name="s")

    @functools.partial(
        pl.kernel, mesh=mesh,
        out_type=jax.ShapeDtypeStruct((_NROW, 4 * _C), jnp.float32),
        scratch_types=[
            pltpu.VMEM((_RPW,), jnp.int32),
            pltpu.VMEM((_CH, 4 * _C), jnp.float32),
            pltpu.SemaphoreType.DMA,
        ],
    )
    def gather_k(ridx_hbm, fquad_hbm, out_hbm, idx_v, buf_v, sem):
        wid = jax.lax.axis_index("s") * 2 + jax.lax.axis_index("c")
        base = wid * _RPW
        pltpu.sync_copy(ridx_hbm.at[pl.ds(base, _RPW)], idx_v)

        def step(g, carry):
            pltpu.async_copy(
                fquad_hbm.at[idx_v.at[pl.ds(g * _CH, _CH)]], buf_v, sem
            ).wait()
            pltpu.sync_copy(buf_v, out_hbm.at[pl.ds(base + g * _CH, _CH)])
            return carry

        jax.lax.fori_loop(0, _RPW // _CH, step, 0)

    return gather_k(ridx, fquad)


def _weight_kernel(g_ref, fx_ref, fy_ref, out_ref):
    q = g_ref[...]
    g00 = q[:, 0 * _C:1 * _C]
    g01 = q[:, 1 * _C:2 * _C]
    g10 = q[:, 2 * _C:3 * _C]
    g11 = q[:, 3 * _C:4 * _C]
    fx = fx_ref[0]
    fy = fy_ref[0]
    out_ref[0] = (g00 * (1.0 - fy) * (1.0 - fx)
                  + g01 * (1.0 - fy) * fx
                  + g10 * fy * (1.0 - fx)
                  + g11 * fy * fx)


def _fc_kernel(x_ref, w1_ref, b1_ref, wc_ref, bc_ref, out_ref):
    h = jax.lax.dot_general(x_ref[...], w1_ref[...], (((1,), (0,)), ((), ())),
                            preferred_element_type=jnp.float32,
                            precision=jax.lax.Precision.DEFAULT)
    h = jnp.maximum(h + b1_ref[...], 0.0)
    out_ref[...] = jax.lax.dot_general(
        h, wc_ref[...], (((1,), (0,)), ((), ())),
        preferred_element_type=jnp.float32,
        precision=jax.lax.Precision.DEFAULT) + bc_ref[...]


def _heads(feature, proposal, W1, b1, Wcls, bcls, Wreg, breg):
    f2 = jnp.zeros((_H * _W + 51, _C), jnp.float32).at[:_H * _W].set(
        feature.transpose(1, 2, 0).reshape(_H * _W, _C))
    fquad = jnp.concatenate(
        [f2[0:2500], f2[1:2501], f2[50:2550], f2[51:2551]], axis=1)
    prop_pad = jnp.zeros((_NPAD, 4), jnp.float32).at[:1000].set(proposal)

    nblk = _OUT * _OUT * (_NPAD // _NB)
    ridx3, fx3, fy3 = pl.pallas_call(
        _idx_kernel,
        grid=(_OUT * _OUT, _NPAD // _NB),
        in_specs=[pl.BlockSpec((_NB, 4), lambda ij, k: (k, 0))],
        out_specs=[
            pl.BlockSpec((1, _NB, 1), lambda ij, k: (ij * 8 + k, 0, 0)),
            pl.BlockSpec((1, _NB, 1), lambda ij, k: (ij * 8 + k, 0, 0)),
            pl.BlockSpec((1, _NB, 1), lambda ij, k: (ij * 8 + k, 0, 0)),
        ],
        out_shape=[
            jax.ShapeDtypeStruct((nblk, _NB, 1), jnp.int32),
            jax.ShapeDtypeStruct((nblk, _NB, 1), jnp.float32),
            jax.ShapeDtypeStruct((nblk, _NB, 1), jnp.float32),
        ],
    )(prop_pad)

    gq = _sc_gather(ridx3.reshape(_NROW), fquad)

    pooled = pl.pallas_call(
        _weight_kernel,
        grid=(_OUT * _OUT, _NPAD // _NB),
        in_specs=[
            pl.BlockSpec((_NB, 4 * _C), lambda ij, k: (ij * 8 + k, 0)),
            pl.BlockSpec((1, _NB, 1), lambda ij, k: (ij * 8 + k, 0, 0)),
            pl.BlockSpec((1, _NB, 1), lambda ij, k: (ij * 8 + k, 0, 0)),
        ],
        out_specs=pl.BlockSpec((1, _NB, _C), lambda ij, k: (ij, k, 0)),
        out_shape=jax.ShapeDtypeStruct((_OUT * _OUT, _NPAD, _C), jnp.float32),
    )(gq, fx3, fy3)

    xmat = pooled.transpose(1, 2, 0).reshape(_NPAD, _K)
    wcat = jnp.concatenate([Wcls, Wreg], axis=1)
    bcat = jnp.concatenate([bcls, breg]).reshape(1, _NHEAD)

    out = pl.pallas_call(
        _fc_kernel,
        grid=(_NPAD // _NB,),
        in_specs=[
            pl.BlockSpec((_NB, _K), lambda k: (k, 0)),
            pl.BlockSpec((_K, 1024), lambda k: (0, 0)),
            pl.BlockSpec((1, 1024), lambda k: (0, 0)),
            pl.BlockSpec((1024, _NHEAD), lambda k: (0, 0)),
            pl.BlockSpec((1, _NHEAD), lambda k: (0, 0)),
        ],
        out_specs=pl.BlockSpec((_NB, _NHEAD), lambda k: (k, 0)),
        out_shape=jax.ShapeDtypeStruct((_NPAD, _NHEAD), jnp.float32),
        compiler_params=pltpu.CompilerParams(
            vmem_limit_bytes=100 * 1024 * 1024),
    )(xmat, W1, b1.reshape(1, 1024), wcat, bcat)
    return out[:1000]


def _nms_kernel(lg_ref, dx_ref, dy_ref, dw_ref, dh_ref, prop_ref, img_ref,
                x1_ref, y1_ref, x2_ref, y2_ref, s_ref, l_ref):
    L, N = lg_ref.shape
    hm1 = img_ref[0:1, 0:1] - 1.0
    wm1 = img_ref[0:1, 1:2] - 1.0

    lg = lg_ref[...]
    m = jnp.max(lg, axis=0, keepdims=True)
    e = jnp.exp(lg - m)
    score = e / jnp.sum(e, axis=0, keepdims=True)

    dx = dx_ref[...] * 0.1
    dy = dy_ref[...] * 0.1
    dw = jnp.minimum(dw_ref[...] * 0.2, _BBOX_CLIP)
    dh = jnp.minimum(dh_ref[...] * 0.2, _BBOX_CLIP)

    px1 = prop_ref[0:1, :]
    py1 = prop_ref[1:2, :]
    px2 = prop_ref[2:3, :]
    py2 = prop_ref[3:4, :]
    w = px2 - px1
    h = py2 - py1
    cx = px1 + 0.5 * w
    cy = py1 + 0.5 * h
    pcx = dx * w + cx
    pcy = dy * h + cy
    pw = jnp.exp(dw) * w
    ph = jnp.exp(dh) * h
    bx1 = jnp.minimum(jnp.maximum(pcx - 0.5 * pw, 0.0), wm1)
    by1 = jnp.minimum(jnp.maximum(pcy - 0.5 * ph, 0.0), hm1)
    bx2 = jnp.minimum(jnp.maximum(pcx + 0.5 * pw, 0.0), wm1)
    by2 = jnp.minimum(jnp.maximum(pcy + 0.5 * ph, 0.0), hm1)
    ws = bx2 - bx1
    hs = by2 - by1
    areas = ws * hs
    lane = jax.lax.broadcasted_iota(jnp.int32, (L, N), 1)
    valid = ((ws >= _MIN_SIZE) & (hs >= _MIN_SIZE) & (score >= _SCORE_TH)
             & (lane < 1000))
    neg = jnp.float32(-1e9)
    s0 = jnp.where(valid, score, neg)

    col = jax.lax.broadcasted_iota(jnp.int32, (L, _NUM_DET), 1)
    zer = jnp.zeros((L, _NUM_DET), jnp.float32)

    def body(t, carry):
        s, os_, ox1, oy1, ox2, oy2 = carry
        mt = jnp.max(s, axis=1, keepdims=True)
        idx = jnp.min(jnp.where(s == mt, lane, jnp.int32(2**30)),
                      axis=1, keepdims=True)
        oh = lane == idx
        sx1 = jnp.sum(jnp.where(oh, bx1, 0.0), axis=1, keepdims=True)
        sy1 = jnp.sum(jnp.where(oh, by1, 0.0), axis=1, keepdims=True)
        sx2 = jnp.sum(jnp.where(oh, bx2, 0.0), axis=1, keepdims=True)
        sy2 = jnp.sum(jnp.where(oh, by2, 0.0), axis=1, keepdims=True)
        xx1 = jnp.maximum(sx1, bx1)
        yy1 = jnp.maximum(sy1, by1)
        xx2 = jnp.minimum(sx2, bx2)
        yy2 = jnp.minimum(sy2, by2)
        inter = (jnp.maximum(xx2 - xx1, 0.0) * jnp.maximum(yy2 - yy1, 0.0))
        ai = (sx2 - sx1) * (sy2 - sy1)
        iou = inter / (ai + areas - inter + 1e-9)
        s = jnp.where(iou > _NMS_TH, neg, s)
        s = jnp.where(oh, neg, s)
        cm = col == t
        os_ = jnp.where(cm, mt, os_)
        ox1 = jnp.where(cm, sx1, ox1)
        oy1 = jnp.where(cm, sy1, oy1)
        ox2 = jnp.where(cm, sx2, ox2)
        oy2 = jnp.where(cm, sy2, oy2)
        return s, os_, ox1, oy1, ox2, oy2

    s, os_, ox1, oy1, ox2, oy2 = jax.lax.fori_loop(
        0, _NUM_DET, body, (s0, zer, zer, zer, zer, zer))

    v = os_ >= _SCORE_TH
    rowl = jax.lax.broadcasted_iota(jnp.int32, (L, _NUM_DET), 0)
    x1_ref[...] = jnp.where(v, ox1, 0.0)
    y1_ref[...] = jnp.where(v, oy1, 0.0)
    x2_ref[...] = jnp.where(v, ox2, 0.0)
    y2_ref[...] = jnp.where(v, oy2, 0.0)
    s_ref[...] = jnp.where(v, os_, 0.0)
    l_ref[...] = jnp.where(v, rowl, 0)


def kernel(feature, proposal, W1, b1, Wcls, bcls, Wreg, breg, image_shape):
    out = _heads(feature, proposal, W1, b1, Wcls, bcls, Wreg, breg)

    def padLN(a, fill=0.0):
        return jnp.full((_LPAD, _NPAD), fill, jnp.float32).at[
            :_NUM_CLASSES, :1000].set(a)

    logitT = padLN(out[:, :_NUM_CLASSES].T, -1e5)
    dxT = padLN(out[:, _NUM_CLASSES + 0::4].T)
    dyT = padLN(out[:, _NUM_CLASSES + 1::4].T)
    dwT = padLN(out[:, _NUM_CLASSES + 2::4].T)
    dhT = padLN(out[:, _NUM_CLASSES + 3::4].T)
    propT = jnp.zeros((8, _NPAD), jnp.float32).at[:4, :1000].set(proposal.T)
    imgf = jnp.asarray(image_shape, jnp.float32).reshape(1, 2)

    shp = (_LPAD, _NUM_DET)
    x1o, y1o, x2o, y2o, so, lo = pl.pallas_call(
        _nms_kernel,
        out_shape=[
            jax.ShapeDtypeStruct(shp, jnp.float32),
            jax.ShapeDtypeStruct(shp, jnp.float32),
            jax.ShapeDtypeStruct(shp, jnp.float32),
            jax.ShapeDtypeStruct(shp, jnp.float32),
            jax.ShapeDtypeStruct(shp, jnp.float32),
            jax.ShapeDtypeStruct(shp, jnp.int32),
        ],
    )(logitT, dxT, dyT, dwT, dhT, propT, imgf)

    boxes = jnp.stack([x1o, y1o, x2o, y2o], axis=-1)[1:_NUM_CLASSES]
    return boxes, so[1:_NUM_CLASSES], lo[1:_NUM_CLASSES]


# speedup vs baseline: 12.3935x; 1.0074x over previous
import functools

import jax
import jax.numpy as jnp
from jax.experimental import pallas as pl
from jax.experimental.pallas import tpu as pltpu
from jax.experimental.pallas import tpu_sc as plsc

_NUM_CLASSES = 21
_OUT = 7
_SCALE = 1.0 / 16.0
_SCORE_TH = 0.05
_NMS_TH = 0.5
_NUM_DET = 50
_MIN_SIZE = 1.0
_BBOX_CLIP = 4.135166556742356
_H = 50
_W = 50
_C = 256
_NB = 128
_NPAD = 1024
_NHEAD = _NUM_CLASSES * 5
_P = 2560
_LPAD = 32
_K = _C * _OUT * _OUT


def _idx_kernel(prop_ref, rx_ref, fx_ref, fy_ref):
    ij = pl.program_id(0)
    i = ij // _OUT
    j = ij - i * _OUT
    gi = (i.astype(jnp.float32) + 0.5) / _OUT
    gj = (j.astype(jnp.float32) + 0.5) / _OUT

    b = prop_ref[...] * _SCALE
    x1 = b[:, 0:1]
    y1 = b[:, 1:2]
    x2 = b[:, 2:3]
    y2 = b[:, 3:4]
    x = jnp.clip(x1 + gj * (x2 - x1), 0.0, _W - 1.0)
    y = jnp.clip(y1 + gi * (y2 - y1), 0.0, _H - 1.0)
    x0 = jnp.clip(jnp.floor(x), 0.0, _W - 2.0)
    y0 = jnp.clip(jnp.floor(y), 0.0, _H - 2.0)
    x0i = x0.astype(jnp.int32)
    y0i = y0.astype(jnp.int32)
    rx_ref[0] = y0i * _W + x0i
    fx_ref[0] = x - x0
    fy_ref[0] = y - y0


_NROW = _OUT * _OUT * _NPAD
_NW = 32
_RPW = _NROW // _NW
_CH = 32


def _sc_gather(ridx, fquad):
    mesh = plsc.VectorSubcoreMesh(core_axis_name="c", subcore_axis_name="s")

    @functools.partial(
        pl.kernel, mesh=mesh,
        out_type=jax.ShapeDtypeStruct((_NROW, 4 * _C), jnp.float32),
        scratch_types=[
            pltpu.VMEM((_RPW,), jnp.int32),
            pltpu.VMEM((_CH, 4 * _C), jnp.float32),
            pltpu.VMEM((_CH, 4 * _C), jnp.float32),
            pltpu.SemaphoreType.DMA,
            pltpu.SemaphoreType.DMA,
        ],
    )
    def gather_k(ridx_hbm, fquad_hbm, out_hbm, idx_v, buf0, buf1, sem0, sem1):
        wid = jax.lax.axis_index("s") * 2 + jax.lax.axis_index("c")
        base = wid * _RPW
        nch = _RPW // _CH
        pltpu.sync_copy(ridx_hbm.at[pl.ds(base, _RPW)], idx_v)

        pltpu.async_copy(fquad_hbm.at[idx_v.at[pl.ds(0, _CH)]], buf0, sem0)

        def step(g, carry):
            buf_cur = [buf0, buf1]
            sem_cur = [sem0, sem1]
            for par in (0, 1):
                @pl.when((g % 2) == par)
                def _():
                    nxt = buf_cur[1 - par]
                    snx = sem_cur[1 - par]

                    @pl.when(g + 1 < nch)
                    def _():
                        pltpu.async_copy(
                            fquad_hbm.at[idx_v.at[pl.ds((g + 1) * _CH, _CH)]],
                            nxt, snx)

                    pltpu.make_async_copy(
                        fquad_hbm.at[pl.ds(0, _CH)], buf_cur[par],
                        sem_cur[par]).wait()
                    pltpu.sync_copy(
                        buf_cur[par], out_hbm.at[pl.ds(base + g * _CH, _CH)])
            return carry

        jax.lax.fori_loop(0, nch, step, 0)

    return gather_k(ridx, fquad)


def _weight_kernel(g_ref, fx_ref, fy_ref, out_ref):
    q = g_ref[...]
    g00 = q[:, 0 * _C:1 * _C]
    g01 = q[:, 1 * _C:2 * _C]
    g10 = q[:, 2 * _C:3 * _C]
    g11 = q[:, 3 * _C:4 * _C]
    fx = fx_ref[0]
    fy = fy_ref[0]
    out_ref[0] = (g00 * (1.0 - fy) * (1.0 - fx)
                  + g01 * (1.0 - fy) * fx
                  + g10 * fy * (1.0 - fx)
                  + g11 * fy * fx)


def _fc_kernel(x_ref, w1_ref, b1_ref, wc_ref, bc_ref, out_ref):
    h = jax.lax.dot_general(x_ref[...], w1_ref[...], (((1,), (0,)), ((), ())),
                            preferred_element_type=jnp.float32,
                            precision=jax.lax.Precision.DEFAULT)
    h = jnp.maximum(h + b1_ref[...], 0.0)
    out_ref[...] = jax.lax.dot_general(
        h, wc_ref[...], (((1,), (0,)), ((), ())),
        preferred_element_type=jnp.float32,
        precision=jax.lax.Precision.DEFAULT) + bc_ref[...]


def _heads(feature, proposal, W1, b1, Wcls, bcls, Wreg, breg):
    f2 = jnp.zeros((_H * _W + 51, _C), jnp.float32).at[:_H * _W].set(
        feature.transpose(1, 2, 0).reshape(_H * _W, _C))
    fquad = jnp.concatenate(
        [f2[0:2500], f2[1:2501], f2[50:2550], f2[51:2551]], axis=1)
    prop_pad = jnp.zeros((_NPAD, 4), jnp.float32).at[:1000].set(proposal)

    nblk = _OUT * _OUT * (_NPAD // _NB)
    ridx3, fx3, fy3 = pl.pallas_call(
        _idx_kernel,
        grid=(_OUT * _OUT, _NPAD // _NB),
        in_specs=[pl.BlockSpec((_NB, 4), lambda ij, k: (k, 0))],
        out_specs=[
            pl.BlockSpec((1, _NB, 1), lambda ij, k: (ij * 8 + k, 0, 0)),
            pl.BlockSpec((1, _NB, 1), lambda ij, k: (ij * 8 + k, 0, 0)),
            pl.BlockSpec((1, _NB, 1), lambda ij, k: (ij * 8 + k, 0, 0)),
        ],
        out_shape=[
            jax.ShapeDtypeStruct((nblk, _NB, 1), jnp.int32),
            jax.ShapeDtypeStruct((nblk, _NB, 1), jnp.float32),
            jax.ShapeDtypeStruct((nblk, _NB, 1), jnp.float32),
        ],
    )(prop_pad)

    gq = _sc_gather(ridx3.reshape(_NROW), fquad)

    pooled = pl.pallas_call(
        _weight_kernel,
        grid=(_OUT * _OUT, _NPAD // _NB),
        in_specs=[
            pl.BlockSpec((_NB, 4 * _C), lambda ij, k: (ij * 8 + k, 0)),
            pl.BlockSpec((1, _NB, 1), lambda ij, k: (ij * 8 + k, 0, 0)),
            pl.BlockSpec((1, _NB, 1), lambda ij, k: (ij * 8 + k, 0, 0)),
        ],
        out_specs=pl.BlockSpec((1, _NB, _C), lambda ij, k: (ij, k, 0)),
        out_shape=jax.ShapeDtypeStruct((_OUT * _OUT, _NPAD, _C), jnp.float32),
    )(gq, fx3, fy3)

    xmat = pooled.transpose(1, 2, 0).reshape(_NPAD, _K)
    wcat = jnp.concatenate([Wcls, Wreg], axis=1)
    bcat = jnp.concatenate([bcls, breg]).reshape(1, _NHEAD)

    out = pl.pallas_call(
        _fc_kernel,
        grid=(_NPAD // _NB,),
        in_specs=[
            pl.BlockSpec((_NB, _K), lambda k: (k, 0)),
            pl.BlockSpec((_K, 1024), lambda k: (0, 0)),
            pl.BlockSpec((1, 1024), lambda k: (0, 0)),
            pl.BlockSpec((1024, _NHEAD), lambda k: (0, 0)),
            pl.BlockSpec((1, _NHEAD), lambda k: (0, 0)),
        ],
        out_specs=pl.BlockSpec((_NB, _NHEAD), lambda k: (k, 0)),
        out_shape=jax.ShapeDtypeStruct((_NPAD, _NHEAD), jnp.float32),
        compiler_params=pltpu.CompilerParams(
            vmem_limit_bytes=100 * 1024 * 1024),
    )(xmat, W1, b1.reshape(1, 1024), wcat, bcat)
    return out[:1000]


def _nms_kernel(lg_ref, dx_ref, dy_ref, dw_ref, dh_ref, prop_ref, img_ref,
                x1_ref, y1_ref, x2_ref, y2_ref, s_ref, l_ref):
    L, N = lg_ref.shape
    hm1 = img_ref[0:1, 0:1] - 1.0
    wm1 = img_ref[0:1, 1:2] - 1.0

    lg = lg_ref[...]
    m = jnp.max(lg, axis=0, keepdims=True)
    e = jnp.exp(lg - m)
    score = e / jnp.sum(e, axis=0, keepdims=True)

    dx = dx_ref[...] * 0.1
    dy = dy_ref[...] * 0.1
    dw = jnp.minimum(dw_ref[...] * 0.2, _BBOX_CLIP)
    dh = jnp.minimum(dh_ref[...] * 0.2, _BBOX_CLIP)

    px1 = prop_ref[0:1, :]
    py1 = prop_ref[1:2, :]
    px2 = prop_ref[2:3, :]
    py2 = prop_ref[3:4, :]
    w = px2 - px1
    h = py2 - py1
    cx = px1 + 0.5 * w
    cy = py1 + 0.5 * h
    pcx = dx * w + cx
    pcy = dy * h + cy
    pw = jnp.exp(dw) * w
    ph = jnp.exp(dh) * h
    bx1 = jnp.minimum(jnp.maximum(pcx - 0.5 * pw, 0.0), wm1)
    by1 = jnp.minimum(jnp.maximum(pcy - 0.5 * ph, 0.0), hm1)
    bx2 = jnp.minimum(jnp.maximum(pcx + 0.5 * pw, 0.0), wm1)
    by2 = jnp.minimum(jnp.maximum(pcy + 0.5 * ph, 0.0), hm1)
    ws = bx2 - bx1
    hs = by2 - by1
    areas = ws * hs
    lane = jax.lax.broadcasted_iota(jnp.int32, (L, N), 1)
    valid = ((ws >= _MIN_SIZE) & (hs >= _MIN_SIZE) & (score >= _SCORE_TH)
             & (lane < 1000))
    neg = jnp.float32(-1e9)
    s0 = jnp.where(valid, score, neg)

    col = jax.lax.broadcasted_iota(jnp.int32, (L, _NUM_DET), 1)
    zer = jnp.zeros((L, _NUM_DET), jnp.float32)

    def body(t, carry):
        s, os_, ox1, oy1, ox2, oy2 = carry
        mt = jnp.max(s, axis=1, keepdims=True)
        idx = jnp.min(jnp.where(s == mt, lane, jnp.int32(2**30)),
                      axis=1, keepdims=True)
        oh = lane == idx
        sx1 = jnp.sum(jnp.where(oh, bx1, 0.0), axis=1, keepdims=True)
        sy1 = jnp.sum(jnp.where(oh, by1, 0.0), axis=1, keepdims=True)
        sx2 = jnp.sum(jnp.where(oh, bx2, 0.0), axis=1, keepdims=True)
        sy2 = jnp.sum(jnp.where(oh, by2, 0.0), axis=1, keepdims=True)
        xx1 = jnp.maximum(sx1, bx1)
        yy1 = jnp.maximum(sy1, by1)
        xx2 = jnp.minimum(sx2, bx2)
        yy2 = jnp.minimum(sy2, by2)
        inter = (jnp.maximum(xx2 - xx1, 0.0) * jnp.maximum(yy2 - yy1, 0.0))
        ai = (sx2 - sx1) * (sy2 - sy1)
        iou = inter / (ai + areas - inter + 1e-9)
        s = jnp.where(iou > _NMS_TH, neg, s)
        s = jnp.where(oh, neg, s)
        cm = col == t
        os_ = jnp.where(cm, mt, os_)
        ox1 = jnp.where(cm, sx1, ox1)
        oy1 = jnp.where(cm, sy1, oy1)
        ox2 = jnp.where(cm, sx2, ox2)
        oy2 = jnp.where(cm, sy2, oy2)
        return s, os_, ox1, oy1, ox2, oy2

    s, os_, ox1, oy1, ox2, oy2 = jax.lax.fori_loop(
        0, _NUM_DET, body, (s0, zer, zer, zer, zer, zer))

    v = os_ >= _SCORE_TH
    rowl = jax.lax.broadcasted_iota(jnp.int32, (L, _NUM_DET), 0)
    x1_ref[...] = jnp.where(v, ox1, 0.0)
    y1_ref[...] = jnp.where(v, oy1, 0.0)
    x2_ref[...] = jnp.where(v, ox2, 0.0)
    y2_ref[...] = jnp.where(v, oy2, 0.0)
    s_ref[...] = jnp.where(v, os_, 0.0)
    l_ref[...] = jnp.where(v, rowl, 0)


def kernel(feature, proposal, W1, b1, Wcls, bcls, Wreg, breg, image_shape):
    out = _heads(feature, proposal, W1, b1, Wcls, bcls, Wreg, breg)

    def padLN(a, fill=0.0):
        return jnp.full((_LPAD, _NPAD), fill, jnp.float32).at[
            :_NUM_CLASSES, :1000].set(a)

    logitT = padLN(out[:, :_NUM_CLASSES].T, -1e5)
    dxT = padLN(out[:, _NUM_CLASSES + 0::4].T)
    dyT = padLN(out[:, _NUM_CLASSES + 1::4].T)
    dwT = padLN(out[:, _NUM_CLASSES + 2::4].T)
    dhT = padLN(out[:, _NUM_CLASSES + 3::4].T)
    propT = jnp.zeros((8, _NPAD), jnp.float32).at[:4, :1000].set(proposal.T)
    imgf = jnp.asarray(image_shape, jnp.float32).reshape(1, 2)

    shp = (_LPAD, _NUM_DET)
    x1o, y1o, x2o, y2o, so, lo = pl.pallas_call(
        _nms_kernel,
        out_shape=[
            jax.ShapeDtypeStruct(shp, jnp.float32),
            jax.ShapeDtypeStruct(shp, jnp.float32),
            jax.ShapeDtypeStruct(shp, jnp.float32),
            jax.ShapeDtypeStruct(shp, jnp.float32),
            jax.ShapeDtypeStruct(shp, jnp.float32),
            jax.ShapeDtypeStruct(shp, jnp.int32),
        ],
    )(logitT, dxT, dyT, dwT, dhT, propT, imgf)

    boxes = jnp.stack([x1o, y1o, x2o, y2o], axis=-1)[1:_NUM_CLASSES]
    return boxes, so[1:_NUM_CLASSES], lo[1:_NUM_CLASSES]
